# baseline (speedup 1.0000x reference)
import functools

import jax
import jax.numpy as jnp
from jax import lax
from jax.experimental import pallas as pl
from jax.experimental.pallas import tpu as pltpu
from jax.experimental.pallas import tpu_sc as plsc

IMS = 8
N = 32
C = 151
R = 101
RP = 128
CP = 152
CH = CP // 2


def _sc_gather_kernel(table_hbm, idx_hbm, out_hbm, idx_v, rows_v, sem):
    info = plsc.get_sparse_core_info()
    nc = info.num_cores
    wid = lax.axis_index("s") * nc + lax.axis_index("c")
    img = wid // 4
    for h in range(2):
        pltpu.sync_copy(idx_hbm.at[2 * wid + h], idx_v)
        pltpu.async_copy(table_hbm.at[idx_v], rows_v, sem).wait()
        pltpu.sync_copy(
            rows_v, out_hbm.at[img, pl.ds((wid % 4) * 8 + h * 4, 4)]
        )


def _sc_gather(table, idx2):
    kern = functools.partial(
        pl.kernel,
        mesh=plsc.VectorSubcoreMesh(core_axis_name="c", subcore_axis_name="s"),
        out_type=jax.ShapeDtypeStruct((IMS, N, CH, RP), jnp.int32),
        scratch_types=[
            pltpu.VMEM((4,), jnp.int32),
            pltpu.VMEM((4, CH, RP), jnp.int32),
            pltpu.SemaphoreType.DMA,
        ],
    )(_sc_gather_kernel)
    return kern(table, idx2)


def _tc_body(g_ref, lrs_ref, lab_ref, out_ref):
    i = pl.program_id(0)

    lrs = lrs_ref[0]
    lsw = jnp.concatenate(
        [lrs[..., 0:1], lrs[..., 51:101], lrs[..., 1:51]], axis=-1
    )
    lswt = jnp.swapaxes(lsw, 0, 1)
    pmat = lrs + lswt
    ix = lax.broadcasted_iota(jnp.int32, (N, N, R), 0)
    iy = lax.broadcasted_iota(jnp.int32, (N, N, R), 1)
    pmat = jnp.where(ix == iy, 0.0, pmat)
    pmat = jnp.concatenate(
        [pmat, jnp.zeros((N, N, RP - R), jnp.float32)], axis=-1
    )

    acc = jnp.zeros((N, CP), dtype=jnp.float32)
    for x in range(N):
        gi = g_ref[0, x]
        gb = pltpu.bitcast(gi, jnp.bfloat16)
        acc = acc + lax.dot_general(
            pmat[x],
            gb.astype(jnp.float32),
            (((1,), (1,)), ((), ())),
            preferred_element_type=jnp.float32,
        )
    acc = acc[:, :C] * 0.5

    m = jnp.max(acc, axis=1, keepdims=True)
    z = acc - m
    lse = jnp.log(jnp.sum(jnp.exp(z), axis=1, keepdims=True))
    ls = z - lse

    lab_col = lab_ref[0]
    iota_c = lax.broadcasted_iota(jnp.int32, (N, C), 1)
    pick = jnp.sum(jnp.where(iota_c == lab_col, ls, 0.0))

    @pl.when(i == 0)
    def _():
        out_ref[...] = jnp.zeros((1, 1), jnp.float32)

    out_ref[...] = out_ref[...] + (-pick) / float(IMS * N)


def kernel(rois, roi_labels, roi_scores, rel_scores, relationship_mat):
    del rois, roi_scores
    lab = roi_labels.astype(jnp.int32)

    ev = relationship_mat[:, 0::2, :]
    od = jnp.pad(relationship_mat[:, 1::2, :], ((0, 0), (0, 1), (0, 0)))
    ev16 = lax.bitcast_convert_type(ev.astype(jnp.bfloat16), jnp.uint16)
    od16 = lax.bitcast_convert_type(od.astype(jnp.bfloat16), jnp.uint16)
    word = ev16.astype(jnp.uint32) | (od16.astype(jnp.uint32) << 16)
    table = lax.bitcast_convert_type(
        jnp.pad(word, ((0, 0), (0, 0), (0, RP - R))), jnp.int32
    )
    g4 = _sc_gather(table, lab.reshape(IMS * N // 4, 4))

    lrs4 = rel_scores.reshape(IMS, N, N, R)
    lab3 = lab.reshape(IMS, N, 1)

    out = pl.pallas_call(
        _tc_body,
        grid=(IMS,),
        in_specs=[
            pl.BlockSpec((1, N, CH, RP), lambda i: (i, 0, 0, 0)),
            pl.BlockSpec((1, N, N, R), lambda i: (i, 0, 0, 0)),
            pl.BlockSpec((1, N, 1), lambda i: (i, 0, 0)),
        ],
        out_specs=pl.BlockSpec((1, 1), lambda i: (0, 0)),
        out_shape=jax.ShapeDtypeStruct((1, 1), jnp.float32),
        compiler_params=pltpu.CompilerParams(
            dimension_semantics=("arbitrary",)
        ),
    )(g4, lrs4, lab3)
    return out[0, 0]


# speedup vs baseline: 2.2063x; 2.2063x over previous
import functools

import jax
import jax.numpy as jnp
from jax import lax
from jax.experimental import pallas as pl
from jax.experimental.pallas import tpu as pltpu
from jax.experimental.pallas import tpu_sc as plsc

IMS = 8
N = 32
C = 151
R = 101
RP = 128
CP = 152
CH = CP // 2


def _sc_gather_kernel(table_hbm, idx_hbm, out_hbm, idx_v, rows_v, sem):
    info = plsc.get_sparse_core_info()
    nc = info.num_cores
    wid = lax.axis_index("s") * nc + lax.axis_index("c")
    img = wid // 4
    for h in range(2):
        pltpu.sync_copy(idx_hbm.at[2 * wid + h], idx_v)
        pltpu.async_copy(table_hbm.at[idx_v], rows_v, sem).wait()
        pltpu.sync_copy(
            rows_v, out_hbm.at[img, pl.ds((wid % 4) * 8 + h * 4, 4)]
        )


def _sc_gather(table, idx2):
    kern = functools.partial(
        pl.kernel,
        mesh=plsc.VectorSubcoreMesh(core_axis_name="c", subcore_axis_name="s"),
        out_type=jax.ShapeDtypeStruct((IMS, N, CH, RP), jnp.int32),
        scratch_types=[
            pltpu.VMEM((4,), jnp.int32),
            pltpu.VMEM((4, CH, RP), jnp.int32),
            pltpu.SemaphoreType.DMA,
        ],
    )(_sc_gather_kernel)
    return kern(table, idx2)


def _pack_body(rm_ref, tab_ref):
    val = rm_ref[...]
    val = jnp.concatenate(
        [val, jnp.zeros((C, C, RP - R), jnp.float32)], axis=-1
    )
    val = jnp.concatenate(
        [val, jnp.zeros((C, CP - C, RP), jnp.float32)], axis=1
    )
    bf = val.astype(jnp.bfloat16)
    tab_ref[...] = pltpu.bitcast(bf, jnp.int32)


def _pack_table(relmat):
    return pl.pallas_call(
        _pack_body,
        out_shape=jax.ShapeDtypeStruct((C, CH, RP), jnp.int32),
    )(relmat)


def _tc_body(g_ref, lrs_ref, lab_ref, out_ref):
    i = pl.program_id(0)

    lrs = lrs_ref[0]
    lsw = jnp.concatenate(
        [lrs[..., 0:1], lrs[..., 51:101], lrs[..., 1:51]], axis=-1
    )
    lswt = jnp.swapaxes(lsw, 0, 1)
    pmat = lrs + lswt
    ix = lax.broadcasted_iota(jnp.int32, (N, N, R), 0)
    iy = lax.broadcasted_iota(jnp.int32, (N, N, R), 1)
    pmat = jnp.where(ix == iy, 0.0, pmat)
    pmat = jnp.concatenate(
        [pmat, jnp.zeros((N, N, RP - R), jnp.float32)], axis=-1
    )

    acc = jnp.zeros((N, CP), dtype=jnp.float32)
    for x in range(N):
        gi = g_ref[0, x]
        gb = pltpu.bitcast(gi, jnp.bfloat16)
        acc = acc + lax.dot_general(
            pmat[x],
            gb.astype(jnp.float32),
            (((1,), (1,)), ((), ())),
            preferred_element_type=jnp.float32,
        )
    acc = acc[:, :C] * 0.5

    m = jnp.max(acc, axis=1, keepdims=True)
    z = acc - m
    lse = jnp.log(jnp.sum(jnp.exp(z), axis=1, keepdims=True))
    ls = z - lse

    lab_col = lab_ref[0]
    iota_c = lax.broadcasted_iota(jnp.int32, (N, C), 1)
    pick = jnp.sum(jnp.where(iota_c == lab_col, ls, 0.0))

    @pl.when(i == 0)
    def _():
        out_ref[...] = jnp.zeros((1, 1), jnp.float32)

    out_ref[...] = out_ref[...] + (-pick) / float(IMS * N)


def kernel(rois, roi_labels, roi_scores, rel_scores, relationship_mat):
    del rois, roi_scores
    lab = roi_labels.astype(jnp.int32)

    table = _pack_table(relationship_mat)
    g4 = _sc_gather(table, lab.reshape(IMS * N // 4, 4))

    lrs4 = rel_scores.reshape(IMS, N, N, R)
    lab3 = lab.reshape(IMS, N, 1)

    out = pl.pallas_call(
        _tc_body,
        grid=(IMS,),
        in_specs=[
            pl.BlockSpec((1, N, CH, RP), lambda i: (i, 0, 0, 0)),
            pl.BlockSpec((1, N, N, R), lambda i: (i, 0, 0, 0)),
            pl.BlockSpec((1, N, 1), lambda i: (i, 0, 0)),
        ],
        out_specs=pl.BlockSpec((1, 1), lambda i: (0, 0)),
        out_shape=jax.ShapeDtypeStruct((1, 1), jnp.float32),
        compiler_params=pltpu.CompilerParams(
            dimension_semantics=("arbitrary",)
        ),
    )(g4, lrs4, lab3)
    return out[0, 0]


# speedup vs baseline: 2.2063x; 1.0000x over previous
import functools

import jax
import jax.numpy as jnp
from jax import lax
from jax.experimental import pallas as pl
from jax.experimental.pallas import tpu as pltpu
from jax.experimental.pallas import tpu_sc as plsc

IMS = 8
N = 32
C = 151
R = 101
RP = 128
CP = 152
CH = CP // 2


def _sc_gather_kernel(table_hbm, idx_hbm, out_hbm, idx_v, rows_v, sem):
    info = plsc.get_sparse_core_info()
    nc = info.num_cores
    wid = lax.axis_index("s") * nc + lax.axis_index("c")
    img = wid // 4
    for h in range(2):
        pltpu.sync_copy(idx_hbm.at[2 * wid + h], idx_v)
        pltpu.async_copy(table_hbm.at[idx_v], rows_v, sem).wait()
        pltpu.sync_copy(
            rows_v, out_hbm.at[img, pl.ds((wid % 4) * 8 + h * 4, 4)]
        )


def _sc_gather(table, idx2):
    kern = functools.partial(
        pl.kernel,
        mesh=plsc.VectorSubcoreMesh(core_axis_name="c", subcore_axis_name="s"),
        out_type=jax.ShapeDtypeStruct((IMS, N, CH, RP), jnp.int32),
        scratch_types=[
            pltpu.VMEM((4,), jnp.int32),
            pltpu.VMEM((4, CH, RP), jnp.int32),
            pltpu.SemaphoreType.DMA,
        ],
    )(_sc_gather_kernel)
    return kern(table, idx2)


def _pack_body(rm_ref, tab_ref):
    val = rm_ref[...]
    val = jnp.concatenate(
        [val, jnp.zeros((C, C, RP - R), jnp.float32)], axis=-1
    )
    val = jnp.concatenate(
        [val, jnp.zeros((C, CP - C, RP), jnp.float32)], axis=1
    )
    bf = val.astype(jnp.bfloat16)
    tab_ref[...] = pltpu.bitcast(bf, jnp.int32)


def _pack_table(relmat):
    return pl.pallas_call(
        _pack_body,
        out_shape=jax.ShapeDtypeStruct((C, CH, RP), jnp.int32),
    )(relmat)


def _tc_body(g_ref, lrs_ref, lab_ref, out_ref):
    i = pl.program_id(0)

    lrs = lrs_ref[0]
    lsw = jnp.concatenate(
        [lrs[..., 0:1], lrs[..., 51:101], lrs[..., 1:51]], axis=-1
    )
    lswt = jnp.swapaxes(lsw, 0, 1)
    pmat = lrs + lswt
    ix = lax.broadcasted_iota(jnp.int32, (N, N, R), 0)
    iy = lax.broadcasted_iota(jnp.int32, (N, N, R), 1)
    pmat = jnp.where(ix == iy, 0.0, pmat)
    pmat = jnp.concatenate(
        [pmat, jnp.zeros((N, N, RP - R), jnp.float32)], axis=-1
    )

    pmat_bf = pmat.astype(jnp.bfloat16)
    acc = jnp.zeros((N, CP), dtype=jnp.float32)
    for x in range(N):
        gi = g_ref[0, x]
        gb = pltpu.bitcast(gi, jnp.bfloat16)
        acc = acc + lax.dot_general(
            pmat_bf[x],
            gb,
            (((1,), (1,)), ((), ())),
            preferred_element_type=jnp.float32,
        )
    acc = acc[:, :C] * 0.5

    m = jnp.max(acc, axis=1, keepdims=True)
    z = acc - m
    lse = jnp.log(jnp.sum(jnp.exp(z), axis=1, keepdims=True))
    ls = z - lse

    lab_col = lab_ref[0]
    iota_c = lax.broadcasted_iota(jnp.int32, (N, C), 1)
    pick = jnp.sum(jnp.where(iota_c == lab_col, ls, 0.0))

    @pl.when(i == 0)
    def _():
        out_ref[...] = jnp.zeros((1, 1), jnp.float32)

    out_ref[...] = out_ref[...] + (-pick) / float(IMS * N)


def kernel(rois, roi_labels, roi_scores, rel_scores, relationship_mat):
    del rois, roi_scores
    lab = roi_labels.astype(jnp.int32)

    table = _pack_table(relationship_mat)
    g4 = _sc_gather(table, lab.reshape(IMS * N // 4, 4))

    lrs4 = rel_scores.reshape(IMS, N, N, R)
    lab3 = lab.reshape(IMS, N, 1)

    out = pl.pallas_call(
        _tc_body,
        grid=(IMS,),
        in_specs=[
            pl.BlockSpec((1, N, CH, RP), lambda i: (i, 0, 0, 0)),
            pl.BlockSpec((1, N, N, R), lambda i: (i, 0, 0, 0)),
            pl.BlockSpec((1, N, 1), lambda i: (i, 0, 0)),
        ],
        out_specs=pl.BlockSpec((1, 1), lambda i: (0, 0)),
        out_shape=jax.ShapeDtypeStruct((1, 1), jnp.float32),
        compiler_params=pltpu.CompilerParams(
            dimension_semantics=("arbitrary",)
        ),
    )(g4, lrs4, lab3)
    return out[0, 0]


# speedup vs baseline: 2.2081x; 1.0008x over previous
import functools

import jax
import jax.numpy as jnp
from jax import lax
from jax.experimental import pallas as pl
from jax.experimental.pallas import tpu as pltpu
from jax.experimental.pallas import tpu_sc as plsc

IMS = 8
N = 32
C = 151
R = 101
RP = 128
CP = 152
CH = CP // 2


def _sc_gather_kernel(table_hbm, idx_hbm, out_hbm, idx_v, rows_v, sem):
    info = plsc.get_sparse_core_info()
    nc = info.num_cores
    wid = lax.axis_index("s") * nc + lax.axis_index("c")
    img = wid // 4
    for h in range(2):
        pltpu.sync_copy(idx_hbm.at[2 * wid + h], idx_v)
        pltpu.async_copy(table_hbm.at[idx_v], rows_v, sem).wait()
        pltpu.sync_copy(
            rows_v, out_hbm.at[img, pl.ds((wid % 4) * 8 + h * 4, 4)]
        )


def _sc_gather(table, idx2):
    kern = functools.partial(
        pl.kernel,
        mesh=plsc.VectorSubcoreMesh(core_axis_name="c", subcore_axis_name="s"),
        out_type=jax.ShapeDtypeStruct((IMS, N, CH, RP), jnp.int32),
        scratch_types=[
            pltpu.VMEM((4,), jnp.int32),
            pltpu.VMEM((4, CH, RP), jnp.int32),
            pltpu.SemaphoreType.DMA,
        ],
    )(_sc_gather_kernel)
    return kern(table, idx2)


def _pack_body(rm_ref, tab_ref):
    val = rm_ref[...]
    val = jnp.concatenate(
        [val, jnp.zeros((C, C, RP - R), jnp.float32)], axis=-1
    )
    val = jnp.concatenate(
        [val, jnp.zeros((C, CP - C, RP), jnp.float32)], axis=1
    )
    bf = val.astype(jnp.bfloat16)
    tab_ref[...] = pltpu.bitcast(bf, jnp.int32)


def _pack_table(relmat):
    return pl.pallas_call(
        _pack_body,
        out_shape=jax.ShapeDtypeStruct((C, CH, RP), jnp.int32),
    )(relmat)


def _tc_body(g_ref, lrs_ref, lab_ref, out_ref):
    i = pl.program_id(0)

    lrs = lrs_ref[0]
    lsw = jnp.concatenate(
        [lrs[..., 0:1], lrs[..., 51:101], lrs[..., 1:51]], axis=-1
    )
    lswt = jnp.swapaxes(lsw, 0, 1)
    pmat = lrs + lswt
    ix = lax.broadcasted_iota(jnp.int32, (N, N, R), 0)
    iy = lax.broadcasted_iota(jnp.int32, (N, N, R), 1)
    pmat = jnp.where(ix == iy, 0.0, pmat)
    pmat = jnp.concatenate(
        [pmat, jnp.zeros((N, N, RP - R), jnp.float32)], axis=-1
    )

    acc = jnp.zeros((N, CP), dtype=jnp.float32)
    for x in range(N):
        gi = g_ref[0, x]
        gb = pltpu.bitcast(gi, jnp.bfloat16)
        acc = acc + lax.dot_general(
            pmat[x],
            gb.astype(jnp.float32),
            (((1,), (1,)), ((), ())),
            preferred_element_type=jnp.float32,
        )
    acc = acc[:, :C] * 0.5

    m = jnp.max(acc, axis=1, keepdims=True)
    z = acc - m
    lse = jnp.log(jnp.sum(jnp.exp(z), axis=1, keepdims=True))
    ls = z - lse

    lab_col = lab_ref[0]
    iota_c = lax.broadcasted_iota(jnp.int32, (N, C), 1)
    pick = jnp.sum(jnp.where(iota_c == lab_col, ls, 0.0))

    @pl.when(i == 0)
    def _():
        out_ref[...] = jnp.zeros((1, 1), jnp.float32)

    out_ref[...] = out_ref[...] + (-pick) / float(IMS * N)


def kernel(rois, roi_labels, roi_scores, rel_scores, relationship_mat):
    del rois, roi_scores
    lab = roi_labels.astype(jnp.int32)

    table = _pack_table(relationship_mat)
    g4 = _sc_gather(table, lab.reshape(IMS * N // 4, 4))

    lrs4 = rel_scores.reshape(IMS, N, N, R)
    lab3 = lab.reshape(IMS, N, 1)

    out = pl.pallas_call(
        _tc_body,
        grid=(IMS,),
        in_specs=[
            pl.BlockSpec((1, N, CH, RP), lambda i: (i, 0, 0, 0)),
            pl.BlockSpec((1, N, N, R), lambda i: (i, 0, 0, 0)),
            pl.BlockSpec((1, N, 1), lambda i: (i, 0, 0)),
        ],
        out_specs=pl.BlockSpec((1, 1), lambda i: (0, 0)),
        out_shape=jax.ShapeDtypeStruct((1, 1), jnp.float32),
        compiler_params=pltpu.CompilerParams(
            dimension_semantics=("arbitrary",)
        ),
    )(g4, lrs4, lab3)
    return out[0, 0]
